# baseline (speedup 1.0000x reference)
import functools

import jax
import jax.numpy as jnp
from jax import lax
from jax.experimental import pallas as pl
from jax.experimental.pallas import tpu as pltpu
from jax.experimental.pallas import tpu_sc as plsc

_L = 16
_NC = 2
_NS = 16


@functools.partial(jax.jit, static_argnums=(3, 4, 5, 6))
def _bilinear_gather(bev_flat, idx, wgt, B, C, N, HW):
    NW = _NC * _NS
    per_w = (B * C) // NW
    w_per_b = NW // B
    n_grp = N // _L
    mesh = plsc.VectorSubcoreMesh(
        core_axis_name="c", subcore_axis_name="s",
        num_cores=_NC, num_subcores=_NS)

    @functools.partial(
        pl.kernel,
        out_type=jax.ShapeDtypeStruct((B, C, N), jnp.float32),
        mesh=mesh,
        compiler_params=pltpu.CompilerParams(needs_layout_passes=False),
        scratch_types=[
            pltpu.VMEM((4, N), jnp.int32),
            pltpu.VMEM((4, N), jnp.float32),
            pltpu.VMEM((HW,), jnp.float32),
            pltpu.VMEM((N,), jnp.float32),
        ],
    )
    def k(bev_hbm, idx_hbm, wgt_hbm, out_hbm, idx_v, wgt_v, img_v, out_v):
        wid = lax.axis_index("s") * _NC + lax.axis_index("c")
        b = wid // w_per_b
        c0 = (wid % w_per_b) * per_w
        pltpu.sync_copy(idx_hbm.at[b], idx_v)
        pltpu.sync_copy(wgt_hbm.at[b], wgt_v)

        def chan_body(j, carry):
            c = c0 + j
            pltpu.sync_copy(bev_hbm.at[b, c], img_v)

            def grp_body(g, carry2):
                s = pl.ds(g * _L, _L)
                acc = plsc.load_gather(img_v, [idx_v[0, s]]) * wgt_v[0, s]
                acc = acc + plsc.load_gather(img_v, [idx_v[1, s]]) * wgt_v[1, s]
                acc = acc + plsc.load_gather(img_v, [idx_v[2, s]]) * wgt_v[2, s]
                acc = acc + plsc.load_gather(img_v, [idx_v[3, s]]) * wgt_v[3, s]
                out_v[s] = acc
                return carry2

            lax.fori_loop(0, n_grp, grp_body, 0)
            pltpu.sync_copy(out_v, out_hbm.at[b, c])
            return carry

        lax.fori_loop(0, per_w, chan_body, 0)

    return k(bev_flat, idx, wgt)


def kernel(keypoints, bev_features, batch_size, bev_stride):
    B, N, _ = keypoints.shape
    _, C, H, W = bev_features.shape

    x = (keypoints[:, :, 0] - 0.0) / 0.05
    y = (keypoints[:, :, 1] - (-40.0)) / 0.05
    x = x / bev_stride
    y = y / bev_stride
    x0 = jnp.floor(x).astype(jnp.int32)
    x1 = x0 + 1
    y0 = jnp.floor(y).astype(jnp.int32)
    y1 = y0 + 1
    x0 = jnp.clip(x0, 0, W - 1)
    x1 = jnp.clip(x1, 0, W - 1)
    y0 = jnp.clip(y0, 0, H - 1)
    y1 = jnp.clip(y1, 0, H - 1)
    x0f = x0.astype(x.dtype)
    x1f = x1.astype(x.dtype)
    y0f = y0.astype(y.dtype)
    y1f = y1.astype(y.dtype)
    wa = (x1f - x) * (y1f - y)
    wb = (x1f - x) * (y - y0f)
    wc = (x - x0f) * (y1f - y)
    wd = (x - x0f) * (y - y0f)
    idx = jnp.stack([y0 * W + x0, y1 * W + x0, y0 * W + x1, y1 * W + x1],
                    axis=1)
    wgt = jnp.stack([wa, wb, wc, wd], axis=1)

    bev_flat = bev_features.reshape(B, C, H * W)
    out_bcn = _bilinear_gather(bev_flat, idx, wgt, B, C, N, H * W)
    out = jnp.transpose(out_bcn, (0, 2, 1))
    batch_scale = jnp.asarray(batch_size, dtype=out.dtype) / B
    return out * batch_scale


# speedup vs baseline: 1.1649x; 1.1649x over previous
import functools

import jax
import jax.numpy as jnp
from jax import lax
from jax.experimental import pallas as pl
from jax.experimental.pallas import tpu as pltpu
from jax.experimental.pallas import tpu_sc as plsc

_L = 16
_NC = 2
_NS = 16


def _bilinear_gather(bev_flat, kp_flat, consts, B, C, N, HW, H, W):
    NW = _NC * _NS
    per_w = (B * C) // NW
    w_per_b = NW // B
    n_grp = N // _L
    mesh = plsc.VectorSubcoreMesh(
        core_axis_name="c", subcore_axis_name="s",
        num_cores=_NC, num_subcores=_NS)

    @functools.partial(
        pl.kernel,
        out_type=jax.ShapeDtypeStruct((B, C, N), jnp.float32),
        mesh=mesh,
        compiler_params=pltpu.CompilerParams(needs_layout_passes=False),
        scratch_types=[
            pltpu.VMEM((N * 3,), jnp.float32),
            pltpu.VMEM((2, _L), jnp.float32),
            pltpu.VMEM((2, N), jnp.int32),
            pltpu.VMEM((4, N), jnp.float32),
            pltpu.VMEM((HW,), jnp.float32),
            pltpu.VMEM((HW,), jnp.float32),
            pltpu.VMEM((N,), jnp.float32),
            pltpu.VMEM((N,), jnp.float32),
            pltpu.SemaphoreType.DMA,
            pltpu.SemaphoreType.DMA,
            pltpu.SemaphoreType.DMA,
            pltpu.SemaphoreType.DMA,
        ],
    )
    def k(bev_hbm, kp_hbm, consts_hbm, out_hbm, kp_v, consts_v, idx_v, wgt_v,
          img0, img1, out0, out1, si0, si1, so0, so1):
        wid = lax.axis_index("s") * _NC + lax.axis_index("c")
        b = wid // w_per_b
        c0 = (wid % w_per_b) * per_w

        pltpu.async_copy(bev_hbm.at[b, c0], img0, si0)
        pltpu.async_copy(bev_hbm.at[b, c0 + 1], img1, si1)
        pltpu.sync_copy(kp_hbm.at[b], kp_v)
        pltpu.sync_copy(consts_hbm, consts_v)
        stride_v = consts_v[0, :]
        scale_v = consts_v[1, :]
        lane3 = lax.iota(jnp.int32, _L) * 3

        def prep_body(g2, carry):
            for u in range(2):
                g = g2 * 2 + u
                pos = lane3 + g * (3 * _L)
                xs = plsc.load_gather(kp_v, [pos])
                ys = plsc.load_gather(kp_v, [pos + 1])
                x = (xs - 0.0) / jnp.float32(0.05) / stride_v
                y = (ys - jnp.float32(-40.0)) / jnp.float32(0.05) / stride_v
                x0t = x.astype(jnp.int32)
                y0t = y.astype(jnp.int32)
                x0c = jnp.clip(x0t, 0, W - 1)
                x1c = jnp.clip(x0t + 1, 0, W - 1)
                y0c = jnp.clip(y0t, 0, H - 1)
                y1c = jnp.clip(y0t + 1, 0, H - 1)
                x0f = x0c.astype(jnp.float32)
                x1f = x1c.astype(jnp.float32)
                y0f = y0c.astype(jnp.float32)
                y1f = y1c.astype(jnp.float32)
                gx = x1f - x
                fx = x - x0f
                gy = y1f - y
                fy = y - y0f
                s = pl.ds(g * _L, _L)
                idx_v[0, s] = y0c * W + x0c
                idx_v[1, s] = (y1c - y0c) * W + (x1c - x0c)
                wgt_v[0, s] = gx * gy * scale_v
                wgt_v[1, s] = gx * fy * scale_v
                wgt_v[2, s] = fx * gy * scale_v
                wgt_v[3, s] = fx * fy * scale_v
            return carry

        lax.fori_loop(0, n_grp // 2, prep_body, 0)

        bufs = ((img0, out0, si0, so0), (img1, out1, si1, so1))

        def chan_body(sidx, carry):
            for u in range(2):
                img, outb, si, so = bufs[u]
                c = c0 + sidx * 2 + u
                pltpu.make_async_copy(bev_hbm.at[b, c], img, si).wait()

                @pl.when(sidx >= 1)
                def _wait_prev_store():
                    pltpu.make_async_copy(outb, out_hbm.at[b, c], so).wait()

                def grp_body(g4, carry2):
                    for v in range(4):
                        g = g4 * 4 + v
                        s = pl.ds(g * _L, _L)
                        ia = idx_v[0, s]
                        code = idx_v[1, s]
                        acc = plsc.load_gather(img, [ia]) * wgt_v[0, s]
                        acc = acc + plsc.load_gather(img, [ia + (code & -2)]) * wgt_v[1, s]
                        acc = acc + plsc.load_gather(img, [ia + (code & 1)]) * wgt_v[2, s]
                        acc = acc + plsc.load_gather(img, [ia + code]) * wgt_v[3, s]
                        outb[s] = acc
                    return carry2

                lax.fori_loop(0, n_grp // 4, grp_body, 0)
                pltpu.async_copy(outb, out_hbm.at[b, c], so)

                @pl.when(c + 2 < c0 + per_w)
                def _prefetch_next():
                    pltpu.async_copy(bev_hbm.at[b, c + 2], img, si)
            return carry

        lax.fori_loop(0, per_w // 2, chan_body, 0)
        pltpu.make_async_copy(out0, out_hbm.at[b, c0], so0).wait()
        pltpu.make_async_copy(out1, out_hbm.at[b, c0], so1).wait()

    return k(bev_flat, kp_flat, consts)


def kernel(keypoints, bev_features, batch_size, bev_stride):
    B, N, _ = keypoints.shape
    _, C, H, W = bev_features.shape
    stride_f = jnp.asarray(bev_stride, jnp.float32)
    scale_f = jnp.asarray(batch_size, jnp.float32) / B
    consts = jnp.stack([jnp.full((_L,), 1.0, jnp.float32) * stride_f,
                        jnp.full((_L,), 1.0, jnp.float32) * scale_f])
    kp_flat = keypoints.reshape(B, N * 3)
    bev_flat = bev_features.reshape(B, C, H * W)
    out_bcn = _bilinear_gather(bev_flat, kp_flat, consts, B, C, N, H * W, H, W)
    return jnp.transpose(out_bcn, (0, 2, 1))
